# Initial kernel scaffold; baseline (speedup 1.0000x reference)
import jax
import jax.numpy as jnp
from jax.experimental import pallas as pl


def kernel(x, y):
    raise NotImplementedError("write your pallas kernel here")


# speedup vs baseline: 1.4477x; 1.4477x over previous
import jax
import jax.numpy as jnp
from jax.experimental import pallas as pl
from jax.experimental.pallas import tpu as pltpu

B, C, S = 8, 1000, 8192
RATIO = 0.2
N_KEEP = int(S * RATIO)
S_BLK = 512
S_GRID = S // S_BLK


def _ce_topk_kernel(x_ref, y_ref, out_ref, l_ref):
    b = pl.program_id(0)
    sb = pl.program_id(1)

    xb = x_ref[0]
    y_row = y_ref[0]
    m = jnp.max(xb, axis=0, keepdims=True)
    ssum = jnp.sum(jnp.exp(xb - m), axis=0, keepdims=True)
    cids = jax.lax.broadcasted_iota(jnp.int32, (C, S_BLK), 0)
    g = jnp.sum(jnp.where(cids == y_row, xb, 0.0), axis=0, keepdims=True)
    l = m + jnp.log(ssum) - g
    l_ref[pl.ds(b, 1), pl.ds(sb * S_BLK, S_BLK)] = l

    @pl.when((b == B - 1) & (sb == S_GRID - 1))
    def _epilogue():
        lv = l_ref[...]
        bits = jax.lax.bitcast_convert_type(lv, jnp.int32)
        ordv = jnp.where(bits < 0, bits ^ jnp.int32(0x7FFFFFFF), bits)
        int_min = jnp.int32(-2147483648)
        p = jnp.sum((ordv >= 0).astype(jnp.int32), axis=1, keepdims=True)
        t0 = jnp.where(p >= N_KEEP, jnp.int32(0), int_min)

        def body(i, t):
            cand = t | (jnp.int32(1) << (30 - i))
            cnt = jnp.sum((ordv >= cand).astype(jnp.int32), axis=1,
                          keepdims=True)
            return jnp.where(cnt >= N_KEEP, cand, t)

        t = jax.lax.fori_loop(0, 31, body, t0)
        vbits = jnp.where(t < 0, t ^ jnp.int32(0x7FFFFFFF), t)
        thr = jax.lax.bitcast_convert_type(vbits, jnp.float32)
        gt = ordv > t
        cnt_gt = jnp.sum(gt.astype(jnp.float32), axis=1, keepdims=True)
        sum_gt = jnp.sum(jnp.where(gt, lv, 0.0), axis=1, keepdims=True)
        row_total = sum_gt + (N_KEEP - cnt_gt) * thr
        out_ref[0, 0] = jnp.sum(row_total) / (N_KEEP * B)


def kernel(x, y):
    out = pl.pallas_call(
        _ce_topk_kernel,
        grid=(B, S_GRID),
        in_specs=[
            pl.BlockSpec((1, C, S_BLK), lambda b, sb: (b, 0, sb)),
            pl.BlockSpec((1, 1, S_BLK), lambda b, sb: (b, 0, sb)),
        ],
        out_specs=pl.BlockSpec((1, 1), lambda b, sb: (0, 0),
                               memory_space=pltpu.SMEM),
        out_shape=jax.ShapeDtypeStruct((1, 1), jnp.float32),
        scratch_shapes=[pltpu.VMEM((B, S), jnp.float32)],
    )(x, y.reshape(B, 1, S).astype(jnp.int32))
    return out[0, 0]
